# baseline (speedup 1.0000x reference)
import functools

import jax
import jax.numpy as jnp
from jax import lax
from jax.experimental import pallas as pl
from jax.experimental.pallas import tpu as pltpu
from jax.experimental.pallas import tpu_sc as plsc

N = 10000
E = 320000
D = 128

NC = 2
NS = 16
NW = NC * NS

C = 128
CPT = 80
E_PAD = NW * CPT * C

ACC_ROWS = 10240
ROWS_PER_TILE = ACC_ROWS // NS
DUMMY = N


def _sc_body(row_hbm, col_hbm, x_hbm, out_hbm, row_all, col_all, buf, acc,
             sem):
    c = lax.axis_index("c")
    s = lax.axis_index("s")
    wid = c * NS + s

    def _zero_row(r, carry):
        for j in range(D // 16):
            buf[r, pl.ds(j * 16, 16)] = jnp.zeros((16,), jnp.float32)
        return carry

    lax.fori_loop(0, C, _zero_row, 0, unroll=False)
    for b in range(ROWS_PER_TILE // C):
        pltpu.sync_copy(buf, acc.at[pl.ds(s * ROWS_PER_TILE + b * C, C)])

    pltpu.sync_copy(row_hbm.at[pl.ds(wid * CPT, CPT)], row_all)
    pltpu.sync_copy(col_hbm.at[pl.ds(wid * CPT, CPT)], col_all)

    def _fix_row(r, carry):
        for j in range(C // 16):
            rv = row_all[r, pl.ds(j * 16, 16)]
            cv = col_all[r, pl.ds(j * 16, 16)]
            row_all[r, pl.ds(j * 16, 16)] = jnp.where(rv == cv, DUMMY, rv)
        return carry

    lax.fori_loop(0, CPT, _fix_row, 0, unroll=False)
    plsc.subcore_barrier()

    def _step(k, carry):
        pltpu.async_copy(x_hbm.at[col_all.at[k]], buf, sem).wait()
        pltpu.sync_copy(buf, acc.at[row_all.at[k]], add=True)
        return carry

    lax.fori_loop(0, CPT, _step, 0, unroll=False)

    plsc.subcore_barrier()

    for b in range(ROWS_PER_TILE // C):
        off = s * ROWS_PER_TILE + b * C
        pltpu.sync_copy(acc.at[pl.ds(off, C)], out_hbm.at[c, pl.ds(off, C)])


_sc_aggregate = functools.partial(
    pl.kernel,
    mesh=plsc.VectorSubcoreMesh(core_axis_name="c", subcore_axis_name="s"),
    out_type=jax.ShapeDtypeStruct((NC, ACC_ROWS, D), jnp.float32),
    scratch_types=[
        pltpu.VMEM((CPT, C), jnp.int32),
        pltpu.VMEM((CPT, C), jnp.int32),
        pltpu.VMEM((C, D), jnp.float32),
        pltpu.VMEM_SHARED((ACC_ROWS, D), jnp.float32),
        pltpu.SemaphoreType.DMA,
    ],
)(_sc_body)


def _mlp_body(x_ref, p_ref, w1_ref, b1_ref, w2_ref, b2_ref, o_ref):
    out = x_ref[...] + p_ref[0] + p_ref[1]
    h = jnp.dot(out, w1_ref[...], preferred_element_type=jnp.float32)
    h = jnp.maximum(h + b1_ref[...], 0.0)
    y = jnp.dot(h, w2_ref[...], preferred_element_type=jnp.float32)
    o_ref[...] = y + b2_ref[...]


MB = 2000


def _mlp(x, partials, W1, b1, W2, b2):
    grid = (N // MB,)
    return pl.pallas_call(
        _mlp_body,
        grid=grid,
        in_specs=[
            pl.BlockSpec((MB, D), lambda i: (i, 0)),
            pl.BlockSpec((NC, MB, D), lambda i: (0, i, 0)),
            pl.BlockSpec((D, D), lambda i: (0, 0)),
            pl.BlockSpec((1, D), lambda i: (0, 0)),
            pl.BlockSpec((D, D), lambda i: (0, 0)),
            pl.BlockSpec((1, D), lambda i: (0, 0)),
        ],
        out_specs=pl.BlockSpec((MB, D), lambda i: (i, 0)),
        out_shape=jax.ShapeDtypeStruct((N, D), jnp.float32),
    )(x, partials, W1, b1.reshape(1, D), W2, b2.reshape(1, D))


def kernel(x, edge_index, W1, b1, W2, b2):
    row = edge_index[0].astype(jnp.int32)
    col = edge_index[1].astype(jnp.int32)
    pad = E_PAD - E
    rowp = jnp.concatenate(
        [row, jnp.full((pad,), DUMMY, jnp.int32)]).reshape(E_PAD // C, C)
    colp = jnp.concatenate(
        [col, jnp.zeros((pad,), jnp.int32)]).reshape(E_PAD // C, C)
    partials = _sc_aggregate(rowp, colp, x)
    return _mlp(x, partials, W1, b1, W2, b2)


# speedup vs baseline: 1.0789x; 1.0789x over previous
import functools

import jax
import jax.numpy as jnp
from jax import lax
from jax.experimental import pallas as pl
from jax.experimental.pallas import tpu as pltpu
from jax.experimental.pallas import tpu_sc as plsc

N = 10000
E = 320000
D = 128

NC = 2
NS = 16
NW = NC * NS

C = 128
CPT = 80
E_PAD = NW * CPT * C

ACC_ROWS = 10240
ROWS_PER_TILE = ACC_ROWS // NS
DUMMY = N


def _sc_body(row_hbm, col_hbm, x_hbm, out_hbm, *scratch):
    row_vs = scratch[0:3]
    col_vs = scratch[3:6]
    bufs = scratch[6:8]
    acc = scratch[8]
    isem = scratch[9:12]
    gsem = scratch[12:14]
    ssem = scratch[14:16]
    c = lax.axis_index("c")
    s = lax.axis_index("s")
    wid = c * NS + s

    def _zero_row(r, carry):
        for j in range(D // 16):
            bufs[0][r, pl.ds(j * 16, 16)] = jnp.zeros((16,), jnp.float32)
        return carry

    lax.fori_loop(0, C, _zero_row, 0, unroll=False)
    for b in range(ROWS_PER_TILE // C):
        pltpu.sync_copy(bufs[0], acc.at[pl.ds(s * ROWS_PER_TILE + b * C, C)])
    plsc.subcore_barrier()

    def _idx_start(k, o):
        base = (wid * CPT + k) * C
        pltpu.async_copy(row_hbm.at[pl.ds(base, C)], row_vs[o], isem[o])
        pltpu.async_copy(col_hbm.at[pl.ds(base, C)], col_vs[o], isem[o])

    def _idx_wait(k, o):
        base = (wid * CPT + k) * C
        pltpu.make_async_copy(row_hbm.at[pl.ds(base, C)], row_vs[o],
                              isem[o]).wait()
        pltpu.make_async_copy(col_hbm.at[pl.ds(base, C)], col_vs[o],
                              isem[o]).wait()

    def _fix(o):
        for j in range(C // 16):
            rv = row_vs[o][pl.ds(j * 16, 16)]
            cv = col_vs[o][pl.ds(j * 16, 16)]
            row_vs[o][pl.ds(j * 16, 16)] = jnp.where(rv == cv, DUMMY, rv)

    def _gather_start(o, b):
        pltpu.async_copy(x_hbm.at[col_vs[o]], bufs[b], gsem[b])

    def _gather_wait(o, b):
        pltpu.make_async_copy(x_hbm.at[col_vs[o]], bufs[b], gsem[b]).wait()

    def _scatter_start(o, b):
        pltpu.async_copy(bufs[b], acc.at[row_vs[o]], ssem[b], add=True)

    def _scatter_wait(o, b):
        pltpu.make_async_copy(bufs[b], acc.at[row_vs[o]], ssem[b]).wait()

    def _step(k, o, b, reuse, issue_next):
        if reuse:
            _scatter_wait((o + 1) % 3, b)
        _idx_wait(k, o)
        _fix(o)
        _gather_start(o, b)
        if issue_next:
            _idx_start(k + 1, (o + 1) % 3)
        _gather_wait(o, b)
        _scatter_start(o, b)

    _idx_start(0, 0)
    _idx_start(1, 1)
    _idx_wait(0, 0)
    _fix(0)
    _gather_start(0, 0)
    _idx_start(2, 2)
    _gather_wait(0, 0)
    _scatter_start(0, 0)
    _step(1, 1, 1, reuse=False, issue_next=False)
    for k in range(2, 6):
        _step(k, k % 3, k % 2, reuse=True, issue_next=True)

    def _six(k6, carry):
        for o6 in range(6):
            k = 6 * k6 + o6
            _step(k, o6 % 3, o6 % 2, reuse=True, issue_next=True)
        return carry

    lax.fori_loop(1, CPT // 6, _six, 0, unroll=False)

    _step(78, 0, 0, reuse=True, issue_next=True)
    _step(79, 1, 1, reuse=True, issue_next=False)
    _scatter_wait(0, 0)
    _scatter_wait(1, 1)

    plsc.subcore_barrier()

    for b in range(ROWS_PER_TILE // C):
        off = s * ROWS_PER_TILE + b * C
        pltpu.sync_copy(acc.at[pl.ds(off, C)], out_hbm.at[c, pl.ds(off, C)])


_sc_aggregate = functools.partial(
    pl.kernel,
    mesh=plsc.VectorSubcoreMesh(core_axis_name="c", subcore_axis_name="s"),
    out_type=jax.ShapeDtypeStruct((NC, ACC_ROWS, D), jnp.float32),
    scratch_types=(
        [pltpu.VMEM((C,), jnp.int32)] * 6
        + [pltpu.VMEM((C, D), jnp.float32)] * 2
        + [pltpu.VMEM_SHARED((ACC_ROWS, D), jnp.float32)]
        + [pltpu.SemaphoreType.DMA] * 7
    ),
)(_sc_body)


def _mlp_body(x_ref, p_ref, w1_ref, b1_ref, w2_ref, b2_ref, o_ref):
    out = x_ref[...] + p_ref[0] + p_ref[1]
    h = jnp.dot(out, w1_ref[...], preferred_element_type=jnp.float32)
    h = jnp.maximum(h + b1_ref[...], 0.0)
    y = jnp.dot(h, w2_ref[...], preferred_element_type=jnp.float32)
    o_ref[...] = y + b2_ref[...]


MB = 2000


def _mlp(x, partials, W1, b1, W2, b2):
    grid = (N // MB,)
    return pl.pallas_call(
        _mlp_body,
        grid=grid,
        in_specs=[
            pl.BlockSpec((MB, D), lambda i: (i, 0)),
            pl.BlockSpec((NC, MB, D), lambda i: (0, i, 0)),
            pl.BlockSpec((D, D), lambda i: (0, 0)),
            pl.BlockSpec((1, D), lambda i: (0, 0)),
            pl.BlockSpec((D, D), lambda i: (0, 0)),
            pl.BlockSpec((1, D), lambda i: (0, 0)),
        ],
        out_specs=pl.BlockSpec((MB, D), lambda i: (i, 0)),
        out_shape=jax.ShapeDtypeStruct((N, D), jnp.float32),
    )(x, partials, W1, b1.reshape(1, D), W2, b2.reshape(1, D))


def kernel(x, edge_index, W1, b1, W2, b2):
    row = edge_index[0].astype(jnp.int32)
    col = edge_index[1].astype(jnp.int32)
    pad = E_PAD - E
    rowp = jnp.concatenate([row, jnp.full((pad,), DUMMY, jnp.int32)])
    colp = jnp.concatenate([col, jnp.zeros((pad,), jnp.int32)])
    partials = _sc_aggregate(rowp, colp, x)
    return _mlp(x, partials, W1, b1, W2, b2)


# speedup vs baseline: 2.4899x; 2.3078x over previous
import functools

import jax
import jax.numpy as jnp
from jax import lax
from jax.experimental import pallas as pl
from jax.experimental.pallas import tpu as pltpu
from jax.experimental.pallas import tpu_sc as plsc

N = 10000
E = 320000
D = 128

NC = 2
NS = 16
NW = NC * NS

C = 128
CHUNKS = E // C
FULL = CHUNKS // NW
REM = CHUNKS % NW

ACC_ROWS = 10240
ROWS_PER_TILE = ACC_ROWS // NS
DUMMY = N


def _sc_body(row_hbm, col_hbm, x_hbm, out_hbm, row_v0, col_v0, row_v1,
             col_v1, buf0, buf1, acc, gsem0, gsem1, ssem0, ssem1):
    c = lax.axis_index("c")
    s = lax.axis_index("s")
    wid = c * NS + s

    def _zero_row(r, carry):
        for j in range(D // 16):
            buf0[r, pl.ds(j * 16, 16)] = jnp.zeros((16,), jnp.float32)
        return carry

    lax.fori_loop(0, C, _zero_row, 0, unroll=False)
    for b in range(ROWS_PER_TILE // C):
        pltpu.sync_copy(buf0, acc.at[pl.ds(s * ROWS_PER_TILE + b * C, C)])
    plsc.subcore_barrier()

    def _fetch_idx(k, row_v, col_v):
        base = k * C
        pltpu.sync_copy(row_hbm.at[pl.ds(base, C)], row_v)
        pltpu.sync_copy(col_hbm.at[pl.ds(base, C)], col_v)
        for j in range(C // 16):
            rv = row_v[pl.ds(j * 16, 16)]
            cv = col_v[pl.ds(j * 16, 16)]
            row_v[pl.ds(j * 16, 16)] = jnp.where(rv == cv, DUMMY, rv)

    def _pair(k2, wid):
        ka = k2 * 2 * NW + wid
        kb = ka + NW
        _fetch_idx(ka, row_v0, col_v0)
        ga = pltpu.async_copy(x_hbm.at[col_v0], buf0, gsem0)
        _fetch_idx(kb, row_v1, col_v1)
        ga.wait()
        sa = pltpu.async_copy(buf0, acc.at[row_v0], ssem0, add=True)
        gb = pltpu.async_copy(x_hbm.at[col_v1], buf1, gsem1)
        gb.wait()
        sb = pltpu.async_copy(buf1, acc.at[row_v1], ssem1, add=True)
        sa.wait()
        sb.wait()
        return wid

    lax.fori_loop(0, FULL // 2, _pair, wid, unroll=False)

    @pl.when(wid < REM)
    def _tail():
        _fetch_idx(FULL * NW + wid, row_v0, col_v0)
        pltpu.async_copy(x_hbm.at[col_v0], buf0, gsem0).wait()
        pltpu.sync_copy(buf0, acc.at[row_v0], add=True)

    plsc.subcore_barrier()

    for b in range(ROWS_PER_TILE // C):
        off = s * ROWS_PER_TILE + b * C
        pltpu.sync_copy(acc.at[pl.ds(off, C)], out_hbm.at[c, pl.ds(off, C)])


_sc_aggregate = functools.partial(
    pl.kernel,
    mesh=plsc.VectorSubcoreMesh(core_axis_name="c", subcore_axis_name="s"),
    out_type=jax.ShapeDtypeStruct((NC, ACC_ROWS, D), jnp.float32),
    scratch_types=[
        pltpu.VMEM((C,), jnp.int32),
        pltpu.VMEM((C,), jnp.int32),
        pltpu.VMEM((C,), jnp.int32),
        pltpu.VMEM((C,), jnp.int32),
        pltpu.VMEM((C, D), jnp.float32),
        pltpu.VMEM((C, D), jnp.float32),
        pltpu.VMEM_SHARED((ACC_ROWS, D), jnp.float32),
        pltpu.SemaphoreType.DMA,
        pltpu.SemaphoreType.DMA,
        pltpu.SemaphoreType.DMA,
        pltpu.SemaphoreType.DMA,
    ],
)(_sc_body)


def _mlp_body(x_ref, p_ref, w1_ref, b1_ref, w2_ref, b2_ref, o_ref):
    out = x_ref[...] + p_ref[0] + p_ref[1]
    h = jnp.dot(out, w1_ref[...], preferred_element_type=jnp.float32)
    h = jnp.maximum(h + b1_ref[...], 0.0)
    y = jnp.dot(h, w2_ref[...], preferred_element_type=jnp.float32)
    o_ref[...] = y + b2_ref[...]


MB = 2000


def _mlp(x, partials, W1, b1, W2, b2):
    grid = (N // MB,)
    return pl.pallas_call(
        _mlp_body,
        grid=grid,
        in_specs=[
            pl.BlockSpec((MB, D), lambda i: (i, 0)),
            pl.BlockSpec((NC, MB, D), lambda i: (0, i, 0)),
            pl.BlockSpec((D, D), lambda i: (0, 0)),
            pl.BlockSpec((1, D), lambda i: (0, 0)),
            pl.BlockSpec((D, D), lambda i: (0, 0)),
            pl.BlockSpec((1, D), lambda i: (0, 0)),
        ],
        out_specs=pl.BlockSpec((MB, D), lambda i: (i, 0)),
        out_shape=jax.ShapeDtypeStruct((N, D), jnp.float32),
    )(x, partials, W1, b1.reshape(1, D), W2, b2.reshape(1, D))


def kernel(x, edge_index, W1, b1, W2, b2):
    row = edge_index[0].astype(jnp.int32)
    col = edge_index[1].astype(jnp.int32)
    partials = _sc_aggregate(row, col, x)
    return _mlp(x, partials, W1, b1, W2, b2)


# speedup vs baseline: 2.7282x; 1.0957x over previous
import functools

import jax
import jax.numpy as jnp
from jax import lax
from jax.experimental import pallas as pl
from jax.experimental.pallas import tpu as pltpu
from jax.experimental.pallas import tpu_sc as plsc

N = 10000
E = 320000
D = 128

NC = 2
NS = 16
NW = NC * NS

C = 128
CHUNKS = E // C
FULL = CHUNKS // NW
REM = CHUNKS % NW

ACC_ROWS = 10112
ROWS_PER_TILE = ACC_ROWS // NS
DUMMY = N


def _sc_body(row_hbm, col_hbm, x_hbm, out_hbm, row_v0, col_v0, row_v1,
             col_v1, row_v2, col_v2, buf0, buf1, buf2, acc,
             gsem0, gsem1, gsem2, ssem0, ssem1, ssem2):
    c = lax.axis_index("c")
    s = lax.axis_index("s")
    wid = c * NS + s

    def _zero_row(r, carry):
        for j in range(D // 16):
            buf0[r, pl.ds(j * 16, 16)] = jnp.zeros((16,), jnp.float32)
        return carry

    lax.fori_loop(0, C, _zero_row, 0, unroll=False)
    for b in range(ROWS_PER_TILE // C):
        pltpu.sync_copy(buf0, acc.at[pl.ds(s * ROWS_PER_TILE + b * C, C)])
    rem_rows = ROWS_PER_TILE % C
    pltpu.sync_copy(
        buf0.at[pl.ds(0, rem_rows)],
        acc.at[pl.ds(s * ROWS_PER_TILE + (ROWS_PER_TILE // C) * C,
                     rem_rows)])
    plsc.subcore_barrier()

    def _fetch_idx(k, row_v, col_v):
        base = k * C
        pltpu.sync_copy(row_hbm.at[pl.ds(base, C)], row_v)
        pltpu.sync_copy(col_hbm.at[pl.ds(base, C)], col_v)
        for j in range(C // 16):
            rv = row_v[pl.ds(j * 16, 16)]
            cv = col_v[pl.ds(j * 16, 16)]
            row_v[pl.ds(j * 16, 16)] = jnp.where(rv == cv, DUMMY, rv)

    def _triplet(k3, wid):
        ka = k3 * 3 * NW + wid
        kb = ka + NW
        kc = kb + NW
        _fetch_idx(ka, row_v0, col_v0)
        ga = pltpu.async_copy(x_hbm.at[col_v0], buf0, gsem0)
        _fetch_idx(kb, row_v1, col_v1)
        ga.wait()
        sa = pltpu.async_copy(buf0, acc.at[row_v0], ssem0, add=True)
        gb = pltpu.async_copy(x_hbm.at[col_v1], buf1, gsem1)
        _fetch_idx(kc, row_v2, col_v2)
        gb.wait()
        sb = pltpu.async_copy(buf1, acc.at[row_v1], ssem1, add=True)
        gc = pltpu.async_copy(x_hbm.at[col_v2], buf2, gsem2)
        gc.wait()
        sc = pltpu.async_copy(buf2, acc.at[row_v2], ssem2, add=True)
        sa.wait()
        sb.wait()
        sc.wait()
        return wid

    lax.fori_loop(0, FULL // 3, _triplet, wid, unroll=False)

    @pl.when(wid < REM)
    def _tail():
        _fetch_idx(FULL * NW + wid, row_v0, col_v0)
        pltpu.async_copy(x_hbm.at[col_v0], buf0, gsem0).wait()
        pltpu.sync_copy(buf0, acc.at[row_v0], add=True)

    plsc.subcore_barrier()

    for b in range(ROWS_PER_TILE // C):
        off = s * ROWS_PER_TILE + b * C
        pltpu.sync_copy(acc.at[pl.ds(off, C)], out_hbm.at[c, pl.ds(off, C)])
    off = s * ROWS_PER_TILE + (ROWS_PER_TILE // C) * C
    pltpu.sync_copy(acc.at[pl.ds(off, rem_rows)],
                    out_hbm.at[c, pl.ds(off, rem_rows)])


_sc_aggregate = functools.partial(
    pl.kernel,
    mesh=plsc.VectorSubcoreMesh(core_axis_name="c", subcore_axis_name="s"),
    out_type=jax.ShapeDtypeStruct((NC, ACC_ROWS, D), jnp.float32),
    scratch_types=[
        pltpu.VMEM((C,), jnp.int32),
        pltpu.VMEM((C,), jnp.int32),
        pltpu.VMEM((C,), jnp.int32),
        pltpu.VMEM((C,), jnp.int32),
        pltpu.VMEM((C,), jnp.int32),
        pltpu.VMEM((C,), jnp.int32),
        pltpu.VMEM((C, D), jnp.float32),
        pltpu.VMEM((C, D), jnp.float32),
        pltpu.VMEM((C, D), jnp.float32),
        pltpu.VMEM_SHARED((ACC_ROWS, D), jnp.float32),
    ] + [pltpu.SemaphoreType.DMA] * 6,
)(_sc_body)


def _mlp_body(x_ref, p_ref, w1_ref, b1_ref, w2_ref, b2_ref, o_ref):
    out = x_ref[...] + p_ref[0] + p_ref[1]
    h = jnp.dot(out, w1_ref[...], preferred_element_type=jnp.float32)
    h = jnp.maximum(h + b1_ref[...], 0.0)
    y = jnp.dot(h, w2_ref[...], preferred_element_type=jnp.float32)
    o_ref[...] = y + b2_ref[...]


MB = 2000


def _mlp(x, partials, W1, b1, W2, b2):
    grid = (N // MB,)
    return pl.pallas_call(
        _mlp_body,
        grid=grid,
        in_specs=[
            pl.BlockSpec((MB, D), lambda i: (i, 0)),
            pl.BlockSpec((NC, MB, D), lambda i: (0, i, 0)),
            pl.BlockSpec((D, D), lambda i: (0, 0)),
            pl.BlockSpec((1, D), lambda i: (0, 0)),
            pl.BlockSpec((D, D), lambda i: (0, 0)),
            pl.BlockSpec((1, D), lambda i: (0, 0)),
        ],
        out_specs=pl.BlockSpec((MB, D), lambda i: (i, 0)),
        out_shape=jax.ShapeDtypeStruct((N, D), jnp.float32),
    )(x, partials, W1, b1.reshape(1, D), W2, b2.reshape(1, D))


def kernel(x, edge_index, W1, b1, W2, b2):
    row = edge_index[0].astype(jnp.int32)
    col = edge_index[1].astype(jnp.int32)
    partials = _sc_aggregate(row, col, x)
    return _mlp(x, partials, W1, b1, W2, b2)


# speedup vs baseline: 2.8864x; 1.0580x over previous
import functools

import jax
import jax.numpy as jnp
from jax import lax
from jax.experimental import pallas as pl
from jax.experimental.pallas import tpu as pltpu
from jax.experimental.pallas import tpu_sc as plsc

N = 10000
E = 320000
D = 128

NC = 2
NS = 16
NW = NC * NS

C = 128
CHUNKS = E // C
FULL = CHUNKS // NW
REM = CHUNKS % NW

ACC_ROWS = 10112
ROWS_PER_TILE = ACC_ROWS // NS
DUMMY = N


def _sc_body(row_hbm, col_hbm, x_hbm, out_hbm, row_v0, col_v0, row_v1,
             col_v1, row_v2, col_v2, buf0, buf1, buf2, acc,
             gsem0, gsem1, gsem2, ssem0, ssem1, ssem2,
             isem0, isem1, isem2):
    c = lax.axis_index("c")
    s = lax.axis_index("s")
    wid = c * NS + s

    def _zero_row(r, carry):
        for j in range(D // 16):
            buf0[r, pl.ds(j * 16, 16)] = jnp.zeros((16,), jnp.float32)
        return carry

    lax.fori_loop(0, C, _zero_row, 0, unroll=False)
    for b in range(ROWS_PER_TILE // C):
        pltpu.sync_copy(buf0, acc.at[pl.ds(s * ROWS_PER_TILE + b * C, C)])
    rem_rows = ROWS_PER_TILE % C
    pltpu.sync_copy(
        buf0.at[pl.ds(0, rem_rows)],
        acc.at[pl.ds(s * ROWS_PER_TILE + (ROWS_PER_TILE // C) * C,
                     rem_rows)])
    plsc.subcore_barrier()

    def _fetch_idx_start(k, row_v, col_v, sem):
        base = k * C
        r = pltpu.async_copy(row_hbm.at[pl.ds(base, C)], row_v, sem)
        q = pltpu.async_copy(col_hbm.at[pl.ds(base, C)], col_v, sem)
        return (r, q)

    def _fetch_idx_finish(descs, row_v, col_v):
        descs[0].wait()
        descs[1].wait()
        for j in range(C // 16):
            rv = row_v[pl.ds(j * 16, 16)]
            cv = col_v[pl.ds(j * 16, 16)]
            row_v[pl.ds(j * 16, 16)] = jnp.where(rv == cv, DUMMY, rv)

    def _fetch_idx(k, row_v, col_v):
        _fetch_idx_finish(_fetch_idx_start(k, row_v, col_v, gsem0),
                          row_v, col_v)

    def _triplet(k3, wid):
        ka = k3 * 3 * NW + wid
        kb = ka + NW
        kc = kb + NW
        ia = _fetch_idx_start(ka, row_v0, col_v0, isem0)
        ib = _fetch_idx_start(kb, row_v1, col_v1, isem1)
        ic = _fetch_idx_start(kc, row_v2, col_v2, isem2)
        _fetch_idx_finish(ia, row_v0, col_v0)
        ga = pltpu.async_copy(x_hbm.at[col_v0], buf0, gsem0)
        _fetch_idx_finish(ib, row_v1, col_v1)
        ga.wait()
        sa = pltpu.async_copy(buf0, acc.at[row_v0], ssem0, add=True)
        gb = pltpu.async_copy(x_hbm.at[col_v1], buf1, gsem1)
        _fetch_idx_finish(ic, row_v2, col_v2)
        gb.wait()
        sb = pltpu.async_copy(buf1, acc.at[row_v1], ssem1, add=True)
        gc = pltpu.async_copy(x_hbm.at[col_v2], buf2, gsem2)
        gc.wait()
        sc = pltpu.async_copy(buf2, acc.at[row_v2], ssem2, add=True)
        sa.wait()
        sb.wait()
        sc.wait()
        return wid

    lax.fori_loop(0, FULL // 3, _triplet, wid, unroll=False)

    @pl.when(wid < REM)
    def _tail():
        _fetch_idx(FULL * NW + wid, row_v0, col_v0)
        pltpu.async_copy(x_hbm.at[col_v0], buf0, gsem0).wait()
        pltpu.sync_copy(buf0, acc.at[row_v0], add=True)

    plsc.subcore_barrier()

    for b in range(ROWS_PER_TILE // C):
        off = s * ROWS_PER_TILE + b * C
        pltpu.sync_copy(acc.at[pl.ds(off, C)], out_hbm.at[c, pl.ds(off, C)])
    off = s * ROWS_PER_TILE + (ROWS_PER_TILE // C) * C
    pltpu.sync_copy(acc.at[pl.ds(off, rem_rows)],
                    out_hbm.at[c, pl.ds(off, rem_rows)])


_sc_aggregate = functools.partial(
    pl.kernel,
    mesh=plsc.VectorSubcoreMesh(core_axis_name="c", subcore_axis_name="s"),
    out_type=jax.ShapeDtypeStruct((NC, ACC_ROWS, D), jnp.float32),
    scratch_types=[
        pltpu.VMEM((C,), jnp.int32),
        pltpu.VMEM((C,), jnp.int32),
        pltpu.VMEM((C,), jnp.int32),
        pltpu.VMEM((C,), jnp.int32),
        pltpu.VMEM((C,), jnp.int32),
        pltpu.VMEM((C,), jnp.int32),
        pltpu.VMEM((C, D), jnp.float32),
        pltpu.VMEM((C, D), jnp.float32),
        pltpu.VMEM((C, D), jnp.float32),
        pltpu.VMEM_SHARED((ACC_ROWS, D), jnp.float32),
    ] + [pltpu.SemaphoreType.DMA] * 9,
)(_sc_body)


def _mlp_body(x_ref, p_ref, w1_ref, b1_ref, w2_ref, b2_ref, o_ref):
    out = x_ref[...] + p_ref[0] + p_ref[1]
    h = jnp.dot(out, w1_ref[...], preferred_element_type=jnp.float32)
    h = jnp.maximum(h + b1_ref[...], 0.0)
    y = jnp.dot(h, w2_ref[...], preferred_element_type=jnp.float32)
    o_ref[...] = y + b2_ref[...]


MB = 2000


def _mlp(x, partials, W1, b1, W2, b2):
    grid = (N // MB,)
    return pl.pallas_call(
        _mlp_body,
        grid=grid,
        in_specs=[
            pl.BlockSpec((MB, D), lambda i: (i, 0)),
            pl.BlockSpec((NC, MB, D), lambda i: (0, i, 0)),
            pl.BlockSpec((D, D), lambda i: (0, 0)),
            pl.BlockSpec((1, D), lambda i: (0, 0)),
            pl.BlockSpec((D, D), lambda i: (0, 0)),
            pl.BlockSpec((1, D), lambda i: (0, 0)),
        ],
        out_specs=pl.BlockSpec((MB, D), lambda i: (i, 0)),
        out_shape=jax.ShapeDtypeStruct((N, D), jnp.float32),
    )(x, partials, W1, b1.reshape(1, D), W2, b2.reshape(1, D))


def kernel(x, edge_index, W1, b1, W2, b2):
    row = edge_index[0].astype(jnp.int32)
    col = edge_index[1].astype(jnp.int32)
    partials = _sc_aggregate(row, col, x)
    return _mlp(x, partials, W1, b1, W2, b2)


# speedup vs baseline: 3.1015x; 1.0745x over previous
import functools

import jax
import jax.numpy as jnp
from jax import lax
from jax.experimental import pallas as pl
from jax.experimental.pallas import tpu as pltpu
from jax.experimental.pallas import tpu_sc as plsc

N = 10000
E = 320000
D = 128

NC = 2
NS = 16
NW = NC * NS

C = 128
CHUNKS = E // C
FULL = CHUNKS // NW
REM = CHUNKS % NW

ACC_ROWS = 10112
ROWS_PER_TILE = ACC_ROWS // NS
DUMMY = N


def _sc_body(row_hbm, col_hbm, x_hbm, out_hbm, row_v0, col_v0, row_v1,
             col_v1, row_v2, col_v2, buf0, buf1, buf2, acc,
             gsem0, gsem1, gsem2, ssem0, ssem1, ssem2,
             isem0, isem1, isem2):
    c = lax.axis_index("c")
    s = lax.axis_index("s")
    wid = c * NS + s

    def _zero_row(r, carry):
        for j in range(D // 16):
            buf0[r, pl.ds(j * 16, 16)] = jnp.zeros((16,), jnp.float32)
        return carry

    lax.fori_loop(0, C, _zero_row, 0, unroll=False)
    for b in range(ROWS_PER_TILE // C):
        pltpu.sync_copy(buf0, acc.at[pl.ds(s * ROWS_PER_TILE + b * C, C)])
    rem_rows = ROWS_PER_TILE % C
    pltpu.sync_copy(
        buf0.at[pl.ds(0, rem_rows)],
        acc.at[pl.ds(s * ROWS_PER_TILE + (ROWS_PER_TILE // C) * C,
                     rem_rows)])
    plsc.subcore_barrier()

    def _fetch_idx_start(k, row_v, col_v, sem):
        base = k * C
        r = pltpu.async_copy(row_hbm.at[pl.ds(base, C)], row_v, sem)
        q = pltpu.async_copy(col_hbm.at[pl.ds(base, C)], col_v, sem)
        return (r, q)

    def _fetch_idx_finish(descs, row_v, col_v):
        descs[0].wait()
        descs[1].wait()
        for j in range(C // 16):
            rv = row_v[pl.ds(j * 16, 16)]
            cv = col_v[pl.ds(j * 16, 16)]
            row_v[pl.ds(j * 16, 16)] = jnp.where(rv == cv, DUMMY, rv)

    def _fetch_idx(k, row_v, col_v):
        _fetch_idx_finish(_fetch_idx_start(k, row_v, col_v, gsem0),
                          row_v, col_v)

    def _sextet(k6, wid):
        ka = k6 * 6 * NW + wid
        kb, kc, kd, ke, kf = (ka + NW, ka + 2 * NW, ka + 3 * NW,
                              ka + 4 * NW, ka + 5 * NW)
        ia = _fetch_idx_start(ka, row_v0, col_v0, isem0)
        ib = _fetch_idx_start(kb, row_v1, col_v1, isem1)
        ic = _fetch_idx_start(kc, row_v2, col_v2, isem2)
        _fetch_idx_finish(ia, row_v0, col_v0)
        ga = pltpu.async_copy(x_hbm.at[col_v0], buf0, gsem0)
        _fetch_idx_finish(ib, row_v1, col_v1)
        ga.wait()
        sa = pltpu.async_copy(buf0, acc.at[row_v0], ssem0, add=True)
        gb = pltpu.async_copy(x_hbm.at[col_v1], buf1, gsem1)
        _fetch_idx_finish(ic, row_v2, col_v2)
        gb.wait()
        sb = pltpu.async_copy(buf1, acc.at[row_v1], ssem1, add=True)
        gc = pltpu.async_copy(x_hbm.at[col_v2], buf2, gsem2)
        sa.wait()
        i_d = _fetch_idx_start(kd, row_v0, col_v0, isem0)
        gc.wait()
        sc = pltpu.async_copy(buf2, acc.at[row_v2], ssem2, add=True)
        _fetch_idx_finish(i_d, row_v0, col_v0)
        gd = pltpu.async_copy(x_hbm.at[col_v0], buf0, gsem0)
        sb.wait()
        ie = _fetch_idx_start(ke, row_v1, col_v1, isem1)
        gd.wait()
        sd = pltpu.async_copy(buf0, acc.at[row_v0], ssem0, add=True)
        _fetch_idx_finish(ie, row_v1, col_v1)
        ge = pltpu.async_copy(x_hbm.at[col_v1], buf1, gsem1)
        sc.wait()
        i_f = _fetch_idx_start(kf, row_v2, col_v2, isem2)
        ge.wait()
        se = pltpu.async_copy(buf1, acc.at[row_v1], ssem1, add=True)
        _fetch_idx_finish(i_f, row_v2, col_v2)
        gf = pltpu.async_copy(x_hbm.at[col_v2], buf2, gsem2)
        gf.wait()
        sf = pltpu.async_copy(buf2, acc.at[row_v2], ssem2, add=True)
        sd.wait()
        se.wait()
        sf.wait()
        return wid

    lax.fori_loop(0, FULL // 6, _sextet, wid, unroll=False)

    @pl.when(wid < REM)
    def _tail():
        _fetch_idx(FULL * NW + wid, row_v0, col_v0)
        pltpu.async_copy(x_hbm.at[col_v0], buf0, gsem0).wait()
        pltpu.sync_copy(buf0, acc.at[row_v0], add=True)

    plsc.subcore_barrier()

    for b in range(ROWS_PER_TILE // C):
        off = s * ROWS_PER_TILE + b * C
        pltpu.sync_copy(acc.at[pl.ds(off, C)], out_hbm.at[c, pl.ds(off, C)])
    off = s * ROWS_PER_TILE + (ROWS_PER_TILE // C) * C
    pltpu.sync_copy(acc.at[pl.ds(off, rem_rows)],
                    out_hbm.at[c, pl.ds(off, rem_rows)])


_sc_aggregate = functools.partial(
    pl.kernel,
    mesh=plsc.VectorSubcoreMesh(core_axis_name="c", subcore_axis_name="s"),
    out_type=jax.ShapeDtypeStruct((NC, ACC_ROWS, D), jnp.float32),
    scratch_types=[
        pltpu.VMEM((C,), jnp.int32),
        pltpu.VMEM((C,), jnp.int32),
        pltpu.VMEM((C,), jnp.int32),
        pltpu.VMEM((C,), jnp.int32),
        pltpu.VMEM((C,), jnp.int32),
        pltpu.VMEM((C,), jnp.int32),
        pltpu.VMEM((C, D), jnp.float32),
        pltpu.VMEM((C, D), jnp.float32),
        pltpu.VMEM((C, D), jnp.float32),
        pltpu.VMEM_SHARED((ACC_ROWS, D), jnp.float32),
    ] + [pltpu.SemaphoreType.DMA] * 9,
)(_sc_body)


def _mlp_body(x_ref, p_ref, w1_ref, b1_ref, w2_ref, b2_ref, o_ref):
    out = x_ref[...] + p_ref[0] + p_ref[1]
    h = jnp.dot(out, w1_ref[...], preferred_element_type=jnp.float32)
    h = jnp.maximum(h + b1_ref[...], 0.0)
    y = jnp.dot(h, w2_ref[...], preferred_element_type=jnp.float32)
    o_ref[...] = y + b2_ref[...]


MB = 2000


def _mlp(x, partials, W1, b1, W2, b2):
    grid = (N // MB,)
    return pl.pallas_call(
        _mlp_body,
        grid=grid,
        in_specs=[
            pl.BlockSpec((MB, D), lambda i: (i, 0)),
            pl.BlockSpec((NC, MB, D), lambda i: (0, i, 0)),
            pl.BlockSpec((D, D), lambda i: (0, 0)),
            pl.BlockSpec((1, D), lambda i: (0, 0)),
            pl.BlockSpec((D, D), lambda i: (0, 0)),
            pl.BlockSpec((1, D), lambda i: (0, 0)),
        ],
        out_specs=pl.BlockSpec((MB, D), lambda i: (i, 0)),
        out_shape=jax.ShapeDtypeStruct((N, D), jnp.float32),
    )(x, partials, W1, b1.reshape(1, D), W2, b2.reshape(1, D))


def kernel(x, edge_index, W1, b1, W2, b2):
    row = edge_index[0].astype(jnp.int32)
    col = edge_index[1].astype(jnp.int32)
    partials = _sc_aggregate(row, col, x)
    return _mlp(x, partials, W1, b1, W2, b2)


# speedup vs baseline: 3.3134x; 1.0683x over previous
import functools

import jax
import jax.numpy as jnp
from jax import lax
from jax.experimental import pallas as pl
from jax.experimental.pallas import tpu as pltpu
from jax.experimental.pallas import tpu_sc as plsc

N = 10000
E = 320000
D = 128

NC = 2
NS = 16
NW = NC * NS

C = 128
CHUNKS = E // C
FULL = CHUNKS // NW
REM = CHUNKS % NW

ACC_ROWS = 10112
ROWS_PER_TILE = ACC_ROWS // NS
DUMMY = N


def _sc_body(ei_hbm, x_hbm, out_hbm, row_v0, col_v0, row_v1,
             col_v1, row_v2, col_v2, buf0, buf1, buf2, acc,
             gsem0, gsem1, gsem2, ssem0, ssem1, ssem2,
             isem0, isem1, isem2):
    c = lax.axis_index("c")
    s = lax.axis_index("s")
    wid = c * NS + s

    def _zero_row(r, carry):
        for j in range(D // 16):
            buf0[r, pl.ds(j * 16, 16)] = jnp.zeros((16,), jnp.float32)
        return carry

    lax.fori_loop(0, C, _zero_row, 0, unroll=False)
    for b in range(ROWS_PER_TILE // C):
        pltpu.sync_copy(buf0, acc.at[pl.ds(s * ROWS_PER_TILE + b * C, C)])
    rem_rows = ROWS_PER_TILE % C
    pltpu.sync_copy(
        buf0.at[pl.ds(0, rem_rows)],
        acc.at[pl.ds(s * ROWS_PER_TILE + (ROWS_PER_TILE // C) * C,
                     rem_rows)])
    plsc.subcore_barrier()

    def _fetch_idx_start(k, row_v, col_v, sem):
        base = k * C
        r = pltpu.async_copy(ei_hbm.at[0, pl.ds(base, C)], row_v, sem)
        q = pltpu.async_copy(ei_hbm.at[1, pl.ds(base, C)], col_v, sem)
        return (r, q)

    def _fetch_idx_finish(descs, row_v, col_v):
        descs[0].wait()
        descs[1].wait()
        for j in range(C // 16):
            rv = row_v[pl.ds(j * 16, 16)]
            cv = col_v[pl.ds(j * 16, 16)]
            row_v[pl.ds(j * 16, 16)] = jnp.where(rv == cv, DUMMY, rv)

    def _fetch_idx(k, row_v, col_v):
        _fetch_idx_finish(_fetch_idx_start(k, row_v, col_v, gsem0),
                          row_v, col_v)

    def _sextet(k6, wid):
        ka = k6 * 6 * NW + wid
        kb, kc, kd, ke, kf = (ka + NW, ka + 2 * NW, ka + 3 * NW,
                              ka + 4 * NW, ka + 5 * NW)
        ia = _fetch_idx_start(ka, row_v0, col_v0, isem0)
        ib = _fetch_idx_start(kb, row_v1, col_v1, isem1)
        ic = _fetch_idx_start(kc, row_v2, col_v2, isem2)
        _fetch_idx_finish(ia, row_v0, col_v0)
        ga = pltpu.async_copy(x_hbm.at[col_v0], buf0, gsem0)
        _fetch_idx_finish(ib, row_v1, col_v1)
        ga.wait()
        sa = pltpu.async_copy(buf0, acc.at[row_v0], ssem0, add=True)
        gb = pltpu.async_copy(x_hbm.at[col_v1], buf1, gsem1)
        _fetch_idx_finish(ic, row_v2, col_v2)
        gb.wait()
        sb = pltpu.async_copy(buf1, acc.at[row_v1], ssem1, add=True)
        gc = pltpu.async_copy(x_hbm.at[col_v2], buf2, gsem2)
        sa.wait()
        i_d = _fetch_idx_start(kd, row_v0, col_v0, isem0)
        gc.wait()
        sc = pltpu.async_copy(buf2, acc.at[row_v2], ssem2, add=True)
        _fetch_idx_finish(i_d, row_v0, col_v0)
        gd = pltpu.async_copy(x_hbm.at[col_v0], buf0, gsem0)
        sb.wait()
        ie = _fetch_idx_start(ke, row_v1, col_v1, isem1)
        gd.wait()
        sd = pltpu.async_copy(buf0, acc.at[row_v0], ssem0, add=True)
        _fetch_idx_finish(ie, row_v1, col_v1)
        ge = pltpu.async_copy(x_hbm.at[col_v1], buf1, gsem1)
        sc.wait()
        i_f = _fetch_idx_start(kf, row_v2, col_v2, isem2)
        ge.wait()
        se = pltpu.async_copy(buf1, acc.at[row_v1], ssem1, add=True)
        _fetch_idx_finish(i_f, row_v2, col_v2)
        gf = pltpu.async_copy(x_hbm.at[col_v2], buf2, gsem2)
        gf.wait()
        sf = pltpu.async_copy(buf2, acc.at[row_v2], ssem2, add=True)
        sd.wait()
        se.wait()
        sf.wait()
        return wid

    lax.fori_loop(0, FULL // 6, _sextet, wid, unroll=False)

    @pl.when(wid < REM)
    def _tail():
        _fetch_idx(FULL * NW + wid, row_v0, col_v0)
        pltpu.async_copy(x_hbm.at[col_v0], buf0, gsem0).wait()
        pltpu.sync_copy(buf0, acc.at[row_v0], add=True)

    plsc.subcore_barrier()

    wds = []
    for b in range(ROWS_PER_TILE // C):
        off = s * ROWS_PER_TILE + b * C
        wds.append(pltpu.async_copy(acc.at[pl.ds(off, C)],
                                    out_hbm.at[c, pl.ds(off, C)], gsem0))
    for wd in wds:
        wd.wait()
    off = s * ROWS_PER_TILE + (ROWS_PER_TILE // C) * C
    pltpu.sync_copy(acc.at[pl.ds(off, rem_rows)],
                    out_hbm.at[c, pl.ds(off, rem_rows)])


_sc_aggregate = functools.partial(
    pl.kernel,
    mesh=plsc.VectorSubcoreMesh(core_axis_name="c", subcore_axis_name="s"),
    out_type=jax.ShapeDtypeStruct((NC, ACC_ROWS, D), jnp.float32),
    scratch_types=[
        pltpu.VMEM((C,), jnp.int32),
        pltpu.VMEM((C,), jnp.int32),
        pltpu.VMEM((C,), jnp.int32),
        pltpu.VMEM((C,), jnp.int32),
        pltpu.VMEM((C,), jnp.int32),
        pltpu.VMEM((C,), jnp.int32),
        pltpu.VMEM((C, D), jnp.float32),
        pltpu.VMEM((C, D), jnp.float32),
        pltpu.VMEM((C, D), jnp.float32),
        pltpu.VMEM_SHARED((ACC_ROWS, D), jnp.float32),
    ] + [pltpu.SemaphoreType.DMA] * 9,
)(_sc_body)


def _xw1_body(x_ref, w1_ref, b1_ref, o_ref):
    o_ref[...] = jnp.dot(x_ref[...], w1_ref[...],
                         preferred_element_type=jnp.float32) + b1_ref[...]


def _xw1(x, W1, b1):
    return pl.pallas_call(
        _xw1_body,
        grid=(N // MB,),
        in_specs=[
            pl.BlockSpec((MB, D), lambda i: (i, 0)),
            pl.BlockSpec((D, D), lambda i: (0, 0)),
            pl.BlockSpec((1, D), lambda i: (0, 0)),
        ],
        out_specs=pl.BlockSpec((MB, D), lambda i: (i, 0)),
        out_shape=jax.ShapeDtypeStruct((N, D), jnp.float32),
    )(x, W1, b1.reshape(1, D))


def _mlp_body(xw1_ref, p_ref, w1_ref, w2_ref, b2_ref, o_ref):
    agg = p_ref[0] + p_ref[1]
    h = jnp.dot(agg, w1_ref[...], preferred_element_type=jnp.float32)
    h = jnp.maximum(h + xw1_ref[...], 0.0)
    y = jnp.dot(h, w2_ref[...], preferred_element_type=jnp.float32)
    o_ref[...] = y + b2_ref[...]


MB = 2000


def _mlp(xw1, partials, W1, W2, b2):
    grid = (N // MB,)
    return pl.pallas_call(
        _mlp_body,
        grid=grid,
        in_specs=[
            pl.BlockSpec((MB, D), lambda i: (i, 0)),
            pl.BlockSpec((NC, MB, D), lambda i: (0, i, 0)),
            pl.BlockSpec((D, D), lambda i: (0, 0)),
            pl.BlockSpec((D, D), lambda i: (0, 0)),
            pl.BlockSpec((1, D), lambda i: (0, 0)),
        ],
        out_specs=pl.BlockSpec((MB, D), lambda i: (i, 0)),
        out_shape=jax.ShapeDtypeStruct((N, D), jnp.float32),
    )(xw1, partials, W1, W2, b2.reshape(1, D))


def kernel(x, edge_index, W1, b1, W2, b2):
    ei = edge_index.astype(jnp.int32)
    partials = _sc_aggregate(ei, x)
    xw1 = _xw1(x, W1, b1)
    return _mlp(xw1, partials, W1, W2, b2)
